# baseline (speedup 1.0000x reference)
import functools

import jax
import jax.numpy as jnp
from jax import lax
from jax.experimental import pallas as pl
from jax.experimental.pallas import tpu as pltpu
from jax.experimental.pallas import tpu_sc as plsc

_E = 16
_K = 2
_ALPHA = 0.01
_LANES = 16
_NW = 32


def _proj_kernel(x_ref, w_ref, scores_ref, aux_ref, acc_s, acc_c, *,
                 n_tokens):
    i = pl.program_id(0)
    nb = pl.num_programs(0)

    x = x_ref[...]
    w = w_ref[...]
    logits = lax.dot_general(
        w, x, (((1,), (1,)), ((), ())),
        preferred_element_type=jnp.float32)
    m = jnp.max(logits, axis=0, keepdims=True)
    ex = jnp.exp(logits - m)
    scores = ex / jnp.sum(ex, axis=0, keepdims=True)
    scores_ref[...] = scores

    iota = lax.broadcasted_iota(jnp.int32, logits.shape, 0)
    i1 = jnp.min(jnp.where(logits == m, iota, _E), axis=0, keepdims=True)
    top1 = iota == i1
    l2 = jnp.where(top1, -jnp.inf, logits)
    m2 = jnp.max(l2, axis=0, keepdims=True)
    i2 = jnp.min(jnp.where(l2 == m2, iota, _E), axis=0, keepdims=True)
    cnt = top1.astype(jnp.float32) + (iota == i2).astype(jnp.float32)

    @pl.when(i == 0)
    def _():
        acc_s[...] = jnp.zeros_like(acc_s)
        acc_c[...] = jnp.zeros_like(acc_c)
    acc_s[...] += jnp.sum(scores, axis=1, keepdims=True)
    acc_c[...] += jnp.sum(cnt, axis=1, keepdims=True)

    @pl.when(i == nb - 1)
    def _():
        scale = _ALPHA * _E / (float(n_tokens) * float(n_tokens) * _K)
        aux_ref[0, 0] = jnp.sum(acc_s[...] * acc_c[...]) * scale


def _topk_sc_kernel(scores_hbm, w_hbm, i_hbm, sv, wv, iv, sem, *, n, tpw):
    wid = lax.axis_index("c") * 16 + lax.axis_index("s")
    base = wid * tpw

    copies = [
        pltpu.make_async_copy(scores_hbm.at[pl.ds(e * n + base, tpw)],
                              sv.at[pl.ds(e * tpw, tpw)], sem)
        for e in range(_E)
    ]
    for c in copies:
        c.start()
    for c in copies:
        c.wait()

    lane = lax.iota(jnp.int32, _LANES)
    pair_lo = lane >> 1
    pair_hi = pair_lo + 8
    odd = (lane & 1) == 1

    dnums = lax.GatherDimensionNumbers(
        offset_dims=(), collapsed_slice_dims=(0,), start_index_map=(0,))

    def permute(a, idx):
        return lax.gather(a, idx[:, None], dnums, (1,),
                          mode=lax.GatherScatterMode.PROMISE_IN_BOUNDS)

    def interleave(a, b):
        lo = jnp.where(odd, permute(b, pair_lo), permute(a, pair_lo))
        hi = jnp.where(odd, permute(b, pair_hi), permute(a, pair_hi))
        return lo, hi

    def body(c, _):
        off = c * _LANES
        b1 = sv[pl.ds(off, _LANES)]
        b2 = jnp.full((_LANES,), -1.0, jnp.float32)
        ix1 = jnp.zeros((_LANES,), jnp.int32)
        ix2 = jnp.zeros((_LANES,), jnp.int32)
        for e in range(1, _E):
            se = sv[pl.ds(e * tpw + off, _LANES)]
            ec = jnp.full((_LANES,), e, jnp.int32)
            beat1 = se > b1
            beat2 = se > b2
            b2 = jnp.where(beat1, b1, jnp.where(beat2, se, b2))
            ix2 = jnp.where(beat1, ix1, jnp.where(beat2, ec, ix2))
            b1 = jnp.where(beat1, se, b1)
            ix1 = jnp.where(beat1, ec, ix1)
        w_lo, w_hi = interleave(b1, b2)
        i_lo, i_hi = interleave(ix1, ix2)
        wv[pl.ds(2 * off, _LANES)] = w_lo
        wv[pl.ds(2 * off + _LANES, _LANES)] = w_hi
        iv[pl.ds(2 * off, _LANES)] = i_lo
        iv[pl.ds(2 * off + _LANES, _LANES)] = i_hi
        return _

    lax.fori_loop(0, tpw // _LANES, body, 0)

    outs = [
        pltpu.make_async_copy(wv, w_hbm.at[pl.ds(_K * base, _K * tpw)], sem),
        pltpu.make_async_copy(iv, i_hbm.at[pl.ds(_K * base, _K * tpw)], sem),
    ]
    for c in outs:
        c.start()
    for c in outs:
        c.wait()


def kernel(hidden_states, weight):
    bsz, seq_len, h = hidden_states.shape
    n = bsz * seq_len
    x = hidden_states.reshape(n, h)

    block = 2048
    nb = n // block

    scores_t, aux = pl.pallas_call(
        functools.partial(_proj_kernel, n_tokens=n),
        grid=(nb,),
        in_specs=[
            pl.BlockSpec((block, h), lambda i: (i, 0)),
            pl.BlockSpec((_E, h), lambda i: (0, 0)),
        ],
        out_specs=[
            pl.BlockSpec((_E, block), lambda i: (0, i)),
            pl.BlockSpec(memory_space=pltpu.SMEM),
        ],
        out_shape=[
            jax.ShapeDtypeStruct((_E, n), jnp.float32),
            jax.ShapeDtypeStruct((1, 1), jnp.float32),
        ],
        scratch_shapes=[
            pltpu.VMEM((_E, 1), jnp.float32),
            pltpu.VMEM((_E, 1), jnp.float32),
        ],
    )(x, weight)

    scores_lin = scores_t.reshape(-1)
    tpw = n // _NW

    mesh = plsc.VectorSubcoreMesh(core_axis_name="c", subcore_axis_name="s",
                                  num_cores=2, num_subcores=16)
    sc = pl.kernel(
        functools.partial(_topk_sc_kernel, n=n, tpw=tpw),
        mesh=mesh,
        out_type=[
            jax.ShapeDtypeStruct((_K * n,), jnp.float32),
            jax.ShapeDtypeStruct((_K * n,), jnp.int32),
        ],
        scratch_types=[
            pltpu.VMEM((_E * tpw,), jnp.float32),
            pltpu.VMEM((_K * tpw,), jnp.float32),
            pltpu.VMEM((_K * tpw,), jnp.int32),
            pltpu.SemaphoreType.DMA,
        ],
    )
    w_lin, i_lin = sc(scores_lin)

    topk_idx = i_lin.reshape(n, _K)
    topk_weight = w_lin.reshape(n, _K)
    return topk_idx, topk_weight, aux[0, 0]


# speedup vs baseline: 1.4179x; 1.4179x over previous
import functools

import jax
import jax.numpy as jnp
from jax import lax
from jax.experimental import pallas as pl
from jax.experimental.pallas import tpu as pltpu
from jax.experimental.pallas import tpu_sc as plsc

_E = 16
_K = 2
_ALPHA = 0.01
_LANES = 16
_NW = 32


def _proj_kernel(x_ref, w_ref, scores_ref, aux_ref, acc_s, acc_c, *,
                 n_tokens):
    i = pl.program_id(0)
    nb = pl.num_programs(0)

    x = x_ref[...]
    w = w_ref[...]
    logits = lax.dot_general(
        w, x, (((1,), (1,)), ((), ())),
        preferred_element_type=jnp.float32)
    m = jnp.max(logits, axis=0, keepdims=True)
    ex = jnp.exp(logits - m)
    scores = ex / jnp.sum(ex, axis=0, keepdims=True)
    scores_ref[...] = scores

    iota = lax.broadcasted_iota(jnp.int32, logits.shape, 0)
    i1 = jnp.min(jnp.where(logits == m, iota, _E), axis=0, keepdims=True)
    top1 = iota == i1
    l2 = jnp.where(top1, -jnp.inf, logits)
    m2 = jnp.max(l2, axis=0, keepdims=True)
    i2 = jnp.min(jnp.where(l2 == m2, iota, _E), axis=0, keepdims=True)
    cnt = top1.astype(jnp.float32) + (iota == i2).astype(jnp.float32)

    @pl.when(i == 0)
    def _():
        acc_s[...] = jnp.zeros_like(acc_s)
        acc_c[...] = jnp.zeros_like(acc_c)
    acc_s[...] += jnp.sum(scores, axis=1, keepdims=True)
    acc_c[...] += jnp.sum(cnt, axis=1, keepdims=True)

    @pl.when(i == nb - 1)
    def _():
        scale = _ALPHA * _E / (float(n_tokens) * float(n_tokens) * _K)
        aux_ref[0, 0] = jnp.sum(acc_s[...] * acc_c[...]) * scale


def _topk_sc_kernel(scores_hbm, w_hbm, i_hbm, sv, wv, iv, sem, *, n, tpw):
    wid = lax.axis_index("c") * 16 + lax.axis_index("s")
    base = wid * tpw

    copies = [
        pltpu.make_async_copy(scores_hbm.at[pl.ds(e * n + base, tpw)],
                              sv.at[pl.ds(e * tpw, tpw)], sem)
        for e in range(_E)
    ]
    for c in copies:
        c.start()
    for c in copies:
        c.wait()

    def body(c, _):
        off = c * _LANES
        b1 = sv[pl.ds(off, _LANES)]
        b2 = jnp.full((_LANES,), -1.0, jnp.float32)
        ix1 = jnp.zeros((_LANES,), jnp.int32)
        ix2 = jnp.zeros((_LANES,), jnp.int32)
        for e in range(1, _E):
            se = sv[pl.ds(e * tpw + off, _LANES)]
            ec = jnp.full((_LANES,), e, jnp.int32)
            beat1 = se > b1
            beat2 = se > b2
            b2 = jnp.where(beat1, b1, jnp.where(beat2, se, b2))
            ix2 = jnp.where(beat1, ix1, jnp.where(beat2, ec, ix2))
            b1 = jnp.where(beat1, se, b1)
            ix1 = jnp.where(beat1, ec, ix1)
        wv[pl.ds(off, _LANES)] = b1
        wv[pl.ds(tpw + off, _LANES)] = b2
        iv[pl.ds(off, _LANES)] = ix1
        iv[pl.ds(tpw + off, _LANES)] = ix2
        return _

    lax.fori_loop(0, tpw // _LANES, body, 0)

    outs = [
        pltpu.make_async_copy(wv.at[pl.ds(0, tpw)],
                              w_hbm.at[pl.ds(base, tpw)], sem),
        pltpu.make_async_copy(wv.at[pl.ds(tpw, tpw)],
                              w_hbm.at[pl.ds(n + base, tpw)], sem),
        pltpu.make_async_copy(iv.at[pl.ds(0, tpw)],
                              i_hbm.at[pl.ds(base, tpw)], sem),
        pltpu.make_async_copy(iv.at[pl.ds(tpw, tpw)],
                              i_hbm.at[pl.ds(n + base, tpw)], sem),
    ]
    for c in outs:
        c.start()
    for c in outs:
        c.wait()


def kernel(hidden_states, weight):
    bsz, seq_len, h = hidden_states.shape
    n = bsz * seq_len
    x = hidden_states.reshape(n, h)

    block = 2048
    nb = n // block

    scores_t, aux = pl.pallas_call(
        functools.partial(_proj_kernel, n_tokens=n),
        grid=(nb,),
        in_specs=[
            pl.BlockSpec((block, h), lambda i: (i, 0)),
            pl.BlockSpec((_E, h), lambda i: (0, 0)),
        ],
        out_specs=[
            pl.BlockSpec((_E, block), lambda i: (0, i)),
            pl.BlockSpec(memory_space=pltpu.SMEM),
        ],
        out_shape=[
            jax.ShapeDtypeStruct((_E, n), jnp.float32),
            jax.ShapeDtypeStruct((1, 1), jnp.float32),
        ],
        scratch_shapes=[
            pltpu.VMEM((_E, 1), jnp.float32),
            pltpu.VMEM((_E, 1), jnp.float32),
        ],
    )(x, weight)

    scores_lin = scores_t.reshape(-1)
    tpw = n // _NW

    mesh = plsc.VectorSubcoreMesh(core_axis_name="c", subcore_axis_name="s",
                                  num_cores=2, num_subcores=16)
    sc = pl.kernel(
        functools.partial(_topk_sc_kernel, n=n, tpw=tpw),
        mesh=mesh,
        out_type=[
            jax.ShapeDtypeStruct((_K * n,), jnp.float32),
            jax.ShapeDtypeStruct((_K * n,), jnp.int32),
        ],
        scratch_types=[
            pltpu.VMEM((_E * tpw,), jnp.float32),
            pltpu.VMEM((_K * tpw,), jnp.float32),
            pltpu.VMEM((_K * tpw,), jnp.int32),
            pltpu.SemaphoreType.DMA,
        ],
    )
    w_lin, i_lin = sc(scores_lin)

    topk_idx = i_lin.reshape(_K, n).T
    topk_weight = w_lin.reshape(_K, n).T
    return topk_idx, topk_weight, aux[0, 0]
